# baseline (speedup 1.0000x reference)
import functools

import jax
import jax.numpy as jnp
from jax import lax
from jax.experimental import pallas as pl
from jax.experimental.pallas import tpu as pltpu
from jax.experimental.pallas import tpu_sc as plsc

_N = 32 * 3 * 512 * 512
_NC = 2
_NS = 16
_NW = _NC * _NS
_PER_W = _N // _NW
_CHUNK = 16384
_NCHUNK = _PER_W // _CHUNK
_LANES = 16


def _mesh():
    return plsc.VectorSubcoreMesh(core_axis_name="c", subcore_axis_name="s")


@functools.partial(
    pl.kernel,
    mesh=_mesh(),
    out_type=[
        jax.ShapeDtypeStruct((_NW * _LANES,), jnp.float32),
        jax.ShapeDtypeStruct((_NW * _LANES,), jnp.int32),
    ],
    scratch_types=[
        pltpu.VMEM((_CHUNK,), jnp.float32),
        pltpu.VMEM((_CHUNK,), jnp.float32),
        pltpu.VMEM((_CHUNK,), jnp.int32),
        pltpu.VMEM((_LANES,), jnp.float32),
        pltpu.VMEM((_LANES,), jnp.int32),
    ],
)
def _masked_l1_sc(hat, obs, mask, out_s, out_c, h_v, o_v, m_v, acc_s_v, acc_c_v):
    wid = lax.axis_index("s") * _NC + lax.axis_index("c")
    base = wid * _PER_W

    def chunk_body(j, carry):
        s, c = carry
        off = base + j * _CHUNK
        pltpu.sync_copy(hat.at[pl.ds(off, _CHUNK)], h_v)
        pltpu.sync_copy(obs.at[pl.ds(off, _CHUNK)], o_v)
        pltpu.sync_copy(mask.at[pl.ds(off, _CHUNK)], m_v)

        def step(i, carry2):
            s2, c2 = carry2
            h = h_v[pl.ds(i * _LANES, _LANES)]
            o = o_v[pl.ds(i * _LANES, _LANES)]
            m = m_v[pl.ds(i * _LANES, _LANES)]
            d = jnp.abs(h - o)
            s2 = s2 + d * m.astype(jnp.float32)
            c2 = c2 + m
            return s2, c2

        return lax.fori_loop(0, _CHUNK // _LANES, step, (s, c))

    s0 = jnp.zeros((_LANES,), jnp.float32)
    c0 = jnp.zeros((_LANES,), jnp.int32)
    s, c = lax.fori_loop(0, _NCHUNK, chunk_body, (s0, c0))

    acc_s_v[...] = s
    acc_c_v[...] = c
    pltpu.sync_copy(acc_s_v, out_s.at[pl.ds(wid * _LANES, _LANES)])
    pltpu.sync_copy(acc_c_v, out_c.at[pl.ds(wid * _LANES, _LANES)])


@jax.jit
def kernel(hat, obs, mask):
    h = hat.reshape(-1)
    o = obs.reshape(-1)
    m = mask.reshape(-1)
    part_s, part_c = _masked_l1_sc(h, o, m)
    return jnp.sum(part_s) / jnp.sum(part_c).astype(jnp.float32)


# speedup vs baseline: 1.4135x; 1.4135x over previous
import functools

import jax
import jax.numpy as jnp
from jax import lax
from jax.experimental import pallas as pl
from jax.experimental.pallas import tpu as pltpu
from jax.experimental.pallas import tpu_sc as plsc

_N = 32 * 3 * 512 * 512
_NC = 2
_NS = 16
_NW = _NC * _NS
_PER_W = _N // _NW
_CHUNK = 16384
_NCHUNK = _PER_W // _CHUNK
_LANES = 16
_NBUF = 2


def _mesh():
    return plsc.VectorSubcoreMesh(core_axis_name="c", subcore_axis_name="s")


@functools.partial(
    pl.kernel,
    mesh=_mesh(),
    out_type=[
        jax.ShapeDtypeStruct((_NW * _LANES,), jnp.float32),
        jax.ShapeDtypeStruct((_NW * _LANES,), jnp.int32),
    ],
    scratch_types=[
        pltpu.VMEM((_NBUF, _CHUNK), jnp.float32),
        pltpu.VMEM((_NBUF, _CHUNK), jnp.float32),
        pltpu.VMEM((_NBUF, _CHUNK), jnp.int32),
        pltpu.VMEM((_LANES,), jnp.float32),
        pltpu.VMEM((_LANES,), jnp.int32),
        pltpu.SemaphoreType.DMA((_NBUF,)),
    ],
)
def _masked_l1_sc(hat, obs, mask, out_s, out_c, h_v, o_v, m_v, acc_s_v, acc_c_v,
                  sems):
    wid = lax.axis_index("s") * _NC + lax.axis_index("c")
    base = wid * _PER_W

    def issue(b, chunk_idx):
        off = base + chunk_idx * _CHUNK
        pltpu.async_copy(hat.at[pl.ds(off, _CHUNK)], h_v.at[b], sems.at[b])
        pltpu.async_copy(obs.at[pl.ds(off, _CHUNK)], o_v.at[b], sems.at[b])
        pltpu.async_copy(mask.at[pl.ds(off, _CHUNK)], m_v.at[b], sems.at[b])

    def drain(b, chunk_idx):
        off = base + chunk_idx * _CHUNK
        pltpu.make_async_copy(hat.at[pl.ds(off, _CHUNK)], h_v.at[b],
                              sems.at[b]).wait()
        pltpu.make_async_copy(obs.at[pl.ds(off, _CHUNK)], o_v.at[b],
                              sems.at[b]).wait()
        pltpu.make_async_copy(mask.at[pl.ds(off, _CHUNK)], m_v.at[b],
                              sems.at[b]).wait()

    issue(0, 0)
    issue(1, 1)

    def pair_body(i, carry):
        j = i * _NBUF

        def one(b, carry2):
            s, c = carry2
            jj = j + b
            drain(b, jj)

            def step(k, carry3):
                s3, c3 = carry3
                h = h_v[b, pl.ds(k * _LANES, _LANES)]
                o = o_v[b, pl.ds(k * _LANES, _LANES)]
                m = m_v[b, pl.ds(k * _LANES, _LANES)]
                d = jnp.abs(h - o)
                s3 = s3 + d * m.astype(jnp.float32)
                c3 = c3 + m
                return s3, c3

            s, c = lax.fori_loop(0, _CHUNK // _LANES, step, (s, c), unroll=8)

            @pl.when(jj + _NBUF < _NCHUNK)
            def _():
                issue(b, jj + _NBUF)

            return s, c

        for b in range(_NBUF):
            carry = one(b, carry)
        return carry

    s0 = jnp.zeros((_LANES,), jnp.float32)
    c0 = jnp.zeros((_LANES,), jnp.int32)
    s, c = lax.fori_loop(0, _NCHUNK // _NBUF, pair_body, (s0, c0))

    acc_s_v[...] = s
    acc_c_v[...] = c
    pltpu.sync_copy(acc_s_v, out_s.at[pl.ds(wid * _LANES, _LANES)])
    pltpu.sync_copy(acc_c_v, out_c.at[pl.ds(wid * _LANES, _LANES)])


@jax.jit
def kernel(hat, obs, mask):
    h = hat.reshape(-1)
    o = obs.reshape(-1)
    m = mask.reshape(-1)
    part_s, part_c = _masked_l1_sc(h, o, m)
    return jnp.sum(part_s) / jnp.sum(part_c).astype(jnp.float32)


# speedup vs baseline: 1.6889x; 1.1948x over previous
import functools

import jax
import jax.numpy as jnp
from jax import lax
from jax.experimental import pallas as pl
from jax.experimental.pallas import tpu as pltpu
from jax.experimental.pallas import tpu_sc as plsc

_N = 32 * 3 * 512 * 512
_NC = 2
_NS = 16
_NW = _NC * _NS
_PER_W = _N // _NW
_CHUNK = 16384
_NCHUNK = _PER_W // _CHUNK
_LANES = 16
_NBUF = 2


def _mesh():
    return plsc.VectorSubcoreMesh(core_axis_name="c", subcore_axis_name="s")


@functools.partial(
    pl.kernel,
    mesh=_mesh(),
    out_type=[
        jax.ShapeDtypeStruct((_NW * _LANES,), jnp.float32),
        jax.ShapeDtypeStruct((_NW * _LANES,), jnp.int32),
    ],
    scratch_types=[
        pltpu.VMEM((_NBUF, _CHUNK), jnp.float32),
        pltpu.VMEM((_NBUF, _CHUNK), jnp.float32),
        pltpu.VMEM((_NBUF, _CHUNK), jnp.int32),
        pltpu.VMEM((_LANES,), jnp.float32),
        pltpu.VMEM((_LANES,), jnp.int32),
        pltpu.SemaphoreType.DMA((_NBUF,)),
    ],
    compiler_params=pltpu.CompilerParams(use_tc_tiling_on_sc=False),
)
def _masked_l1_sc(hat, obs, mask, out_s, out_c, h_v, o_v, m_v, acc_s_v, acc_c_v,
                  sems):
    wid = lax.axis_index("s") * _NC + lax.axis_index("c")
    base = wid * _PER_W

    def issue(b, chunk_idx):
        off = base + chunk_idx * _CHUNK
        pltpu.async_copy(hat.at[pl.ds(off, _CHUNK)], h_v.at[b], sems.at[b])
        pltpu.async_copy(obs.at[pl.ds(off, _CHUNK)], o_v.at[b], sems.at[b])
        pltpu.async_copy(mask.at[pl.ds(off, _CHUNK)], m_v.at[b], sems.at[b])

    def drain(b, chunk_idx):
        off = base + chunk_idx * _CHUNK
        pltpu.make_async_copy(hat.at[pl.ds(off, _CHUNK)], h_v.at[b],
                              sems.at[b]).wait()
        pltpu.make_async_copy(obs.at[pl.ds(off, _CHUNK)], o_v.at[b],
                              sems.at[b]).wait()
        pltpu.make_async_copy(mask.at[pl.ds(off, _CHUNK)], m_v.at[b],
                              sems.at[b]).wait()

    issue(0, 0)
    issue(1, 1)

    def pair_body(i, carry):
        j = i * _NBUF

        def one(b, carry2):
            s, c = carry2
            jj = j + b
            drain(b, jj)

            def step(k, carry3):
                s3, c3 = carry3
                h = h_v[b, pl.ds(k * _LANES, _LANES)]
                o = o_v[b, pl.ds(k * _LANES, _LANES)]
                m = m_v[b, pl.ds(k * _LANES, _LANES)]
                d = jnp.abs(h - o)
                s3 = s3 + d * m.astype(jnp.float32)
                c3 = c3 + m
                return s3, c3

            s, c = lax.fori_loop(0, _CHUNK // _LANES, step, (s, c), unroll=8)

            @pl.when(jj + _NBUF < _NCHUNK)
            def _():
                issue(b, jj + _NBUF)

            return s, c

        for b in range(_NBUF):
            carry = one(b, carry)
        return carry

    s0 = jnp.zeros((_LANES,), jnp.float32)
    c0 = jnp.zeros((_LANES,), jnp.int32)
    s, c = lax.fori_loop(0, _NCHUNK // _NBUF, pair_body, (s0, c0))

    acc_s_v[...] = s
    acc_c_v[...] = c
    pltpu.sync_copy(acc_s_v, out_s.at[pl.ds(wid * _LANES, _LANES)])
    pltpu.sync_copy(acc_c_v, out_c.at[pl.ds(wid * _LANES, _LANES)])


@jax.jit
def kernel(hat, obs, mask):
    h = hat.reshape(-1)
    o = obs.reshape(-1)
    m = mask.reshape(-1)
    part_s, part_c = _masked_l1_sc(h, o, m)
    return jnp.sum(part_s) / jnp.sum(part_c).astype(jnp.float32)


# speedup vs baseline: 4.4129x; 2.6129x over previous
import functools

import jax
import jax.numpy as jnp
from jax import lax
from jax.experimental import pallas as pl
from jax.experimental.pallas import tpu as pltpu
from jax.experimental.pallas import tpu_sc as plsc

_B = 32
_C = 3
_H = 512
_W = 512
_NC = 2
_NS = 16
_NW = _NC * _NS
_ROWS = 32
_CHUNKS_PER_CH = _H // _ROWS
_NCHUNK = _C * _CHUNKS_PER_CH
_LANES = 16
_NBUF = 2
_VECS = _ROWS * _W // _LANES


def _mesh():
    return plsc.VectorSubcoreMesh(core_axis_name="c", subcore_axis_name="s")


@functools.partial(
    pl.kernel,
    mesh=_mesh(),
    out_type=[
        jax.ShapeDtypeStruct((_NW * _LANES,), jnp.float32),
        jax.ShapeDtypeStruct((_NW * _LANES,), jnp.int32),
    ],
    scratch_types=[
        pltpu.VMEM((_NBUF, _ROWS, _W), jnp.float32),
        pltpu.VMEM((_NBUF, _ROWS, _W), jnp.float32),
        pltpu.VMEM((_NBUF, _ROWS, _W), jnp.int32),
        pltpu.VMEM((_LANES,), jnp.float32),
        pltpu.VMEM((_LANES,), jnp.int32),
        pltpu.SemaphoreType.DMA((_NBUF,)),
    ],
)
def _masked_l1_sc(hat, obs, mask, out_s, out_c, h_v, o_v, m_v, acc_s_v, acc_c_v,
                  sems):
    wid = lax.axis_index("s") * _NC + lax.axis_index("c")

    def chunk_slices(chunk_idx):
        ch = lax.shift_right_logical(chunk_idx, 4)
        r0 = lax.mul(lax.bitwise_and(chunk_idx, 15), _ROWS)
        return ch, r0

    def issue(b, chunk_idx):
        ch, r0 = chunk_slices(chunk_idx)
        pltpu.async_copy(hat.at[wid, ch, pl.ds(r0, _ROWS), :], h_v.at[b],
                         sems.at[b])
        pltpu.async_copy(obs.at[wid, ch, pl.ds(r0, _ROWS), :], o_v.at[b],
                         sems.at[b])
        pltpu.async_copy(mask.at[wid, ch, pl.ds(r0, _ROWS), :], m_v.at[b],
                         sems.at[b])

    def drain(b, chunk_idx):
        ch, r0 = chunk_slices(chunk_idx)
        pltpu.make_async_copy(hat.at[wid, ch, pl.ds(r0, _ROWS), :], h_v.at[b],
                              sems.at[b]).wait()
        pltpu.make_async_copy(obs.at[wid, ch, pl.ds(r0, _ROWS), :], o_v.at[b],
                              sems.at[b]).wait()
        pltpu.make_async_copy(mask.at[wid, ch, pl.ds(r0, _ROWS), :], m_v.at[b],
                              sems.at[b]).wait()

    issue(0, 0)
    issue(1, 1)

    def pair_body(i, carry):
        j = i * _NBUF

        def one(b, carry2):
            s, c = carry2
            jj = j + b
            drain(b, jj)

            def step(k, carry3):
                s3, c3 = carry3
                r = lax.shift_right_logical(k, 5)
                col = lax.mul(lax.bitwise_and(k, 31), _LANES)
                h = h_v[b, r, pl.ds(col, _LANES)]
                o = o_v[b, r, pl.ds(col, _LANES)]
                m = m_v[b, r, pl.ds(col, _LANES)]
                d = jnp.abs(h - o)
                s3 = s3 + d * m.astype(jnp.float32)
                c3 = c3 + m
                return s3, c3

            s, c = lax.fori_loop(0, _VECS, step, (s, c), unroll=8)

            @pl.when(jj + _NBUF < _NCHUNK)
            def _():
                issue(b, jj + _NBUF)

            return s, c

        for b in range(_NBUF):
            carry = one(b, carry)
        return carry

    s0 = jnp.zeros((_LANES,), jnp.float32)
    c0 = jnp.zeros((_LANES,), jnp.int32)
    s, c = lax.fori_loop(0, _NCHUNK // _NBUF, pair_body, (s0, c0))

    acc_s_v[...] = s
    acc_c_v[...] = c
    pltpu.sync_copy(acc_s_v, out_s.at[pl.ds(wid * _LANES, _LANES)])
    pltpu.sync_copy(acc_c_v, out_c.at[pl.ds(wid * _LANES, _LANES)])


@jax.jit
def kernel(hat, obs, mask):
    part_s, part_c = _masked_l1_sc(hat, obs, mask)
    return jnp.sum(part_s) / jnp.sum(part_c).astype(jnp.float32)


# speedup vs baseline: 5.1532x; 1.1677x over previous
import functools

import jax
import jax.numpy as jnp
from jax import lax
from jax.experimental import pallas as pl
from jax.experimental.pallas import tpu as pltpu
from jax.experimental.pallas import tpu_sc as plsc

_B = 32
_C = 3
_H = 512
_W = 512
_BT = 22
_BS = _B - _BT
_NC = 2
_NS = 16
_NW = _NC * _NS
_ROWS = 32
_CHUNKS_PER_SLAB = _C * (_H // _ROWS)
_NCHUNK = _BS * _CHUNKS_PER_SLAB
_Q = _NCHUNK // _NW
assert _Q * _NW == _NCHUNK
_LANES = 16
_NBUF = 2
_VECS = _ROWS * _W // _LANES


def _mesh():
    return plsc.VectorSubcoreMesh(core_axis_name="c", subcore_axis_name="s")


@functools.partial(
    pl.kernel,
    mesh=_mesh(),
    out_type=[
        jax.ShapeDtypeStruct((_NW * _LANES,), jnp.float32),
        jax.ShapeDtypeStruct((_NW * _LANES,), jnp.int32),
    ],
    scratch_types=[
        pltpu.VMEM((_NBUF, _ROWS, _W), jnp.float32),
        pltpu.VMEM((_NBUF, _ROWS, _W), jnp.float32),
        pltpu.VMEM((_NBUF, _ROWS, _W), jnp.int32),
        pltpu.VMEM((_LANES,), jnp.float32),
        pltpu.VMEM((_LANES,), jnp.int32),
        pltpu.SemaphoreType.DMA((_NBUF,)),
    ],
)
def _masked_l1_sc(hat, obs, mask, out_s, out_c, h_v, o_v, m_v, acc_s_v, acc_c_v,
                  sems):
    wid = lax.axis_index("s") * _NC + lax.axis_index("c")
    g0 = wid * _Q

    def chunk_slices(local_idx):
        g = g0 + local_idx
        slab = lax.div(g, _CHUNKS_PER_SLAB)
        rem = lax.rem(g, _CHUNKS_PER_SLAB)
        b_idx = _BT + slab
        ch = lax.shift_right_logical(rem, 4)
        r0 = lax.mul(lax.bitwise_and(rem, 15), _ROWS)
        return b_idx, ch, r0

    def issue(b, local_idx):
        bi, ch, r0 = chunk_slices(local_idx)
        pltpu.async_copy(hat.at[bi, ch, pl.ds(r0, _ROWS), :], h_v.at[b],
                         sems.at[b])
        pltpu.async_copy(obs.at[bi, ch, pl.ds(r0, _ROWS), :], o_v.at[b],
                         sems.at[b])
        pltpu.async_copy(mask.at[bi, ch, pl.ds(r0, _ROWS), :], m_v.at[b],
                         sems.at[b])

    def drain(b, local_idx):
        bi, ch, r0 = chunk_slices(local_idx)
        pltpu.make_async_copy(hat.at[bi, ch, pl.ds(r0, _ROWS), :], h_v.at[b],
                              sems.at[b]).wait()
        pltpu.make_async_copy(obs.at[bi, ch, pl.ds(r0, _ROWS), :], o_v.at[b],
                              sems.at[b]).wait()
        pltpu.make_async_copy(mask.at[bi, ch, pl.ds(r0, _ROWS), :], m_v.at[b],
                              sems.at[b]).wait()

    issue(0, 0)
    issue(1, 1)

    def pair_body(i, carry):
        j = i * _NBUF

        def one(b, carry2):
            s, c = carry2
            jj = j + b
            drain(b, jj)

            def step(k, carry3):
                s3, c3 = carry3
                r = lax.shift_right_logical(k, 5)
                col = lax.mul(lax.bitwise_and(k, 31), _LANES)
                h = h_v[b, r, pl.ds(col, _LANES)]
                o = o_v[b, r, pl.ds(col, _LANES)]
                m = m_v[b, r, pl.ds(col, _LANES)]
                d = jnp.abs(h - o)
                s3 = s3 + d * m.astype(jnp.float32)
                c3 = c3 + m
                return s3, c3

            s, c = lax.fori_loop(0, _VECS, step, (s, c), unroll=8)

            @pl.when(jj + _NBUF < _Q)
            def _():
                issue(b, jj + _NBUF)

            return s, c

        for b in range(_NBUF):
            carry = one(b, carry)
        return carry

    s0 = jnp.zeros((_LANES,), jnp.float32)
    c0 = jnp.zeros((_LANES,), jnp.int32)
    s, c = lax.fori_loop(0, _Q // _NBUF, pair_body, (s0, c0))

    acc_s_v[...] = s
    acc_c_v[...] = c
    pltpu.sync_copy(acc_s_v, out_s.at[pl.ds(wid * _LANES, _LANES)])
    pltpu.sync_copy(acc_c_v, out_c.at[pl.ds(wid * _LANES, _LANES)])


def _tc_body(h_ref, o_ref, m_ref, out_s_ref, out_c_ref, acc_s, acc_c):
    i = pl.program_id(0)

    @pl.when(i == 0)
    def _():
        acc_s[0] = 0.0
        acc_c[0] = 0

    h = h_ref[0]
    o = o_ref[0]
    m = m_ref[0]
    d = jnp.abs(h - o) * m.astype(jnp.float32)
    acc_s[0] += jnp.sum(d)
    acc_c[0] += jnp.sum(m)

    @pl.when(i == _BT - 1)
    def _():
        out_s_ref[0] = acc_s[0]
        out_c_ref[0] = acc_c[0]


_tc_part = pl.pallas_call(
    _tc_body,
    grid=(_BT,),
    in_specs=[
        pl.BlockSpec((1, _C, _H, _W), lambda i: (i, 0, 0, 0)),
        pl.BlockSpec((1, _C, _H, _W), lambda i: (i, 0, 0, 0)),
        pl.BlockSpec((1, _C, _H, _W), lambda i: (i, 0, 0, 0)),
    ],
    out_specs=[
        pl.BlockSpec(memory_space=pltpu.SMEM),
        pl.BlockSpec(memory_space=pltpu.SMEM),
    ],
    out_shape=[
        jax.ShapeDtypeStruct((1,), jnp.float32),
        jax.ShapeDtypeStruct((1,), jnp.int32),
    ],
    scratch_shapes=[
        pltpu.SMEM((1,), jnp.float32),
        pltpu.SMEM((1,), jnp.int32),
    ],
)


@jax.jit
def kernel(hat, obs, mask):
    part_s, part_c = _masked_l1_sc(hat, obs, mask)
    tc_s, tc_c = _tc_part(hat, obs, mask)
    total_s = jnp.sum(part_s) + tc_s[0]
    total_c = jnp.sum(part_c) + tc_c[0]
    return total_s / total_c.astype(jnp.float32)
